# Initial kernel scaffold; baseline (speedup 1.0000x reference)
import jax
import jax.numpy as jnp
from jax.experimental import pallas as pl


def kernel(input, c_0, W0, b0, W1, b1, W2, b2, W3, b3):
    raise NotImplementedError("write your pallas kernel here")


# speedup vs baseline: 1.1309x; 1.1309x over previous
import functools

import jax
import jax.numpy as jnp
from jax.experimental import pallas as pl
from jax.experimental.pallas import tpu as pltpu

NUM_LAYERS = 4
D = 2048
BATCH = 4096
TILE_B = 256


def _layer_kernel(x_ref, c0_ref, w_ref, b_ref, h_ref, c1_ref):
    x32 = x_ref[...]
    xb = x32.astype(jnp.bfloat16)
    u = jnp.dot(xb, w_ref[...], preferred_element_type=jnp.float32)
    x_tilde = u[:, :D]
    f = jax.nn.sigmoid(u[:, D:2 * D] + b_ref[0, :])
    r = jax.nn.sigmoid(u[:, 2 * D:] + b_ref[1, :])
    c1 = f * c0_ref[...] + (1.0 - f) * x_tilde
    h_ref[...] = r * jnp.tanh(c1) + (1.0 - r) * x32
    c1_ref[...] = c1


@functools.partial(jax.jit, static_argnames=())
def _layer(x, c0, w_bf16, b2):
    nb = BATCH // TILE_B
    return pl.pallas_call(
        _layer_kernel,
        grid=(nb,),
        in_specs=[
            pl.BlockSpec((TILE_B, D), lambda i: (i, 0)),
            pl.BlockSpec((TILE_B, D), lambda i: (i, 0)),
            pl.BlockSpec((D, 3 * D), lambda i: (0, 0)),
            pl.BlockSpec((2, D), lambda i: (0, 0)),
        ],
        out_specs=[
            pl.BlockSpec((TILE_B, D), lambda i: (i, 0)),
            pl.BlockSpec((TILE_B, D), lambda i: (i, 0)),
        ],
        out_shape=[
            jax.ShapeDtypeStruct((BATCH, D), jnp.float32),
            jax.ShapeDtypeStruct((BATCH, D), jnp.float32),
        ],
        compiler_params=pltpu.CompilerParams(
            dimension_semantics=("arbitrary",),
        ),
    )(x, c0, w_bf16, b2)


def kernel(input, c_0, W0, b0, W1, b1, W2, b2, W3, b3):
    Ws = [W0, W1, W2, W3]
    bs = [b0, b1, b2, b3]
    h = input
    c1_list = []
    for i in range(NUM_LAYERS):
        w16 = Ws[i].astype(jnp.bfloat16)
        b2d = bs[i].reshape(2, D)
        h, c1 = _layer(h, c_0[i], w16, b2d)
        c1_list.append(c1)
    return (h, jnp.stack(c1_list))
